# speedup vs baseline: 3.2683x; 3.2683x over previous
import jax
import jax.numpy as jnp
from jax.experimental import pallas as pl

_S_BLK = 256


def _add_kernel(x_ref, pos_ref, out_ref):
    out_ref[...] = x_ref[...] + pos_ref[...][None, :, :]


def kernel(x, pos_embedding):
    B, S, D = x.shape
    grid = (S // _S_BLK,)
    return pl.pallas_call(
        _add_kernel,
        grid=grid,
        in_specs=[
            pl.BlockSpec((B, _S_BLK, D), lambda i: (0, i, 0)),
            pl.BlockSpec((_S_BLK, D), lambda i: (i, 0)),
        ],
        out_specs=pl.BlockSpec((B, _S_BLK, D), lambda i: (0, i, 0)),
        out_shape=jax.ShapeDtypeStruct((B, S, D), x.dtype),
    )(x, pos_embedding[:S])
